# baseline (speedup 1.0000x reference)
import functools

import jax
import jax.numpy as jnp
from jax import lax
from jax.experimental import pallas as pl
from jax.experimental.pallas import tpu as pltpu
from jax.experimental.pallas import tpu_sc as plsc

DIM_IN = 2048
DIM_OUT = 2048
NUM_SEGMENTS = 16
N_TOK = 4096

TM = 256
NB = N_TOK // TM
SEG_PER_GROUP = 8
GRID_G = NB + SEG_PER_GROUP - 1

_NC = 2
_NS = 16
_NW = _NC * _NS
_ROWS_PER_W = N_TOK // _NW
_CH = 32
_NCHUNK = _ROWS_PER_W // _CH


def _row_gather(table, idx):
    ncols = table.shape[1]
    mesh = plsc.VectorSubcoreMesh(core_axis_name="c", subcore_axis_name="s")

    @functools.partial(
        pl.kernel,
        mesh=mesh,
        out_type=jax.ShapeDtypeStruct((N_TOK, ncols), jnp.float32),
        scratch_types=[
            pltpu.VMEM((_CH,), jnp.int32),
            pltpu.VMEM((_CH, ncols), jnp.float32),
            pltpu.SemaphoreType.DMA,
        ],
    )
    def k(table_hbm, idx_hbm, out_hbm, idx_v, rows_v, sem):
        wid = lax.axis_index("s") * _NC + lax.axis_index("c")
        base = wid * _ROWS_PER_W
        for c in range(_NCHUNK):
            off = base + c * _CH
            pltpu.sync_copy(idx_hbm.at[pl.ds(off, _CH)], idx_v)
            pltpu.async_copy(table_hbm.at[idx_v], rows_v, sem).wait()
            pltpu.sync_copy(rows_v, out_hbm.at[pl.ds(off, _CH)])

    return k(table, idx)


def _wpipe(eidl_ref, first_ref, nxt_ref, slot_ref, boot_ref, w_hbm,
           wbuf, wb16, sems, t):
    @pl.when((t == 0) & (boot_ref[0] == 1))
    def _():
        pltpu.make_async_copy(
            w_hbm.at[boot_ref[1]], wbuf.at[0], sems.at[0]).start()

    @pl.when(first_ref[t] == 1)
    def _():
        slot = slot_ref[t]
        pltpu.make_async_copy(
            w_hbm.at[eidl_ref[t]], wbuf.at[slot], sems.at[slot]).wait()

        @pl.when(nxt_ref[t] >= 0)
        def _():
            pltpu.make_async_copy(
                w_hbm.at[nxt_ref[t]], wbuf.at[1 - slot], sems.at[1 - slot]
            ).start()

        wb16[...] = wbuf[slot].astype(jnp.bfloat16)


def _tile_acc(x_ref, wb16, b_ref, bid_ref, rs, re, t):
    base = bid_ref[t] * TM
    rows = base + lax.broadcasted_iota(jnp.int32, (TM, 1), 0)
    mask = (rows >= rs) & (rows < re)
    acc = lax.dot_general(
        x_ref[...].astype(jnp.bfloat16),
        wb16[...],
        dimension_numbers=(((1,), (1,)), ((), ())),
        preferred_element_type=jnp.float32,
    )
    return mask, acc + b_ref[0]


def _gmm0_body(eidl_ref, bid_ref, rs_ref, re_ref, first_ref, nxt_ref,
               slot_ref, cpf_ref, boot_ref,
               x_ref, w_hbm, b_ref, o_ref, wbuf, wb16, sems):
    t = pl.program_id(0)
    rs = rs_ref[t]
    re = re_ref[t]
    _wpipe(eidl_ref, first_ref, nxt_ref, slot_ref, boot_ref, w_hbm,
           wbuf, wb16, sems, t)

    @pl.when(rs < re)
    def _():
        mask, lin = _tile_acc(x_ref, wb16, b_ref, bid_ref, rs, re, t)
        o_ref[...] = jnp.where(mask, lin, o_ref[...])


def _gmm1_body(eidl_ref, bid_ref, rs_ref, re_ref, first_ref, nxt_ref,
               slot_ref, cpf_ref, boot_ref, fv_ref,
               x_ref, w_hbm, b_ref, p_ref, o_ref, wbuf, wb16, sems):
    t = pl.program_id(0)
    rs = rs_ref[t]
    re = re_ref[t]
    _wpipe(eidl_ref, first_ref, nxt_ref, slot_ref, boot_ref, w_hbm,
           wbuf, wb16, sems, t)

    @pl.when(rs < re)
    def _():
        mask, lin = _tile_acc(x_ref, wb16, b_ref, bid_ref, rs, re, t)
        prev = jnp.where(fv_ref[t] == 1, p_ref[...], o_ref[...])
        o_ref[...] = jnp.where(mask, lin, prev)

    @pl.when(cpf_ref[t] == 1)
    def _():
        o_ref[...] = p_ref[...]


def _group_meta(starts, ends, counts, lo):
    i32 = jnp.int32
    seg_ids = jnp.arange(lo, lo + SEG_PER_GROUP, dtype=i32)
    gcounts = lax.dynamic_slice(counts, (lo,), (SEG_PER_GROUP,))
    gstarts = lax.dynamic_slice(starts, (lo,), (SEG_PER_GROUP,))
    gends = lax.dynamic_slice(ends, (lo,), (SEG_PER_GROUP,))
    ne = gcounts > 0
    pos = jnp.where(ne, jnp.cumsum(ne.astype(i32)) - 1, SEG_PER_GROUP)
    big = i32(N_TOK + TM)
    cstarts = jnp.full((SEG_PER_GROUP,), big, i32).at[pos].set(
        gstarts, mode="drop")
    cends = jnp.full((SEG_PER_GROUP,), big, i32).at[pos].set(
        gends, mode="drop")
    ceid = jnp.zeros((SEG_PER_GROUP,), i32).at[pos].set(seg_ids, mode="drop")

    b = jnp.arange(NB, dtype=i32)
    first_idx = jnp.searchsorted(cends, b * TM, side="right").astype(i32)
    last_idx = jnp.searchsorted(cstarts, (b + 1) * TM, side="left").astype(i32)
    nb_tiles = last_idx - first_idx
    steps_pb = jnp.maximum(nb_tiles, 1)
    csteps = jnp.cumsum(steps_pb)
    stepstart = csteps - steps_pb
    total = csteps[-1]

    t = jnp.arange(GRID_G, dtype=i32)
    bt = jnp.minimum(jnp.searchsorted(csteps, t, side="right"),
                     NB - 1).astype(i32)
    j = t - stepstart[bt]
    is_skip = t >= total
    has_tiles = nb_tiles[bt] > 0
    is_copy = (~is_skip) & (~has_tiles)
    is_real = (~is_skip) & has_tiles
    gi = jnp.clip(first_idx[bt] + j, 0, SEG_PER_GROUP - 1)
    eid = ceid[gi]
    rs = jnp.where(is_real, cstarts[gi], 0).astype(i32)
    re = jnp.where(is_real, jnp.minimum(cends[gi], N_TOK), 0).astype(i32)
    bid = jnp.where(is_skip, NB - 1, bt).astype(i32)
    fv = (j == 0).astype(i32)

    filled = lax.cummax(jnp.where(is_real, eid, -1), axis=0)
    filled_prev = jnp.concatenate([jnp.full((1,), -1, i32), filled[:-1]])
    first = (is_real & (eid != filled_prev)).astype(i32)
    ordinal = jnp.cumsum(first) - 1
    slot = (ordinal % 2).astype(i32)
    n_distinct = jnp.sum(first)
    opos = jnp.where(first == 1, ordinal, GRID_G)
    order_e = jnp.zeros((GRID_G,), i32).at[opos].set(eid - lo, mode="drop")
    nxt = jnp.where(ordinal + 1 < n_distinct,
                    order_e[jnp.clip(ordinal + 1, 0, GRID_G - 1)],
                    -1).astype(i32)
    eidl = jnp.where(is_real, eid - lo, order_e[0]).astype(i32)
    boot = jnp.stack([(n_distinct > 0).astype(i32), order_e[0]])
    cpf = is_copy.astype(i32)
    return eidl, bid, rs, re, first, nxt, slot, cpf, boot, fv


def _specs(nargs):
    def xmap(t, *s):
        return (s[1][t], 0)

    def wmap(t, *s):
        return (s[0][t], 0, 0)

    in_specs = [
        pl.BlockSpec((TM, DIM_IN), xmap),
        pl.BlockSpec(memory_space=pl.ANY),
        pl.BlockSpec((1, 1, DIM_OUT), lambda t, *s: (s[0][t], 0, 0)),
    ]
    if nargs == 4:
        in_specs.append(pl.BlockSpec((TM, DIM_OUT), xmap))
    return in_specs, pl.BlockSpec((TM, DIM_OUT), xmap)


def _scratch():
    return [
        pltpu.VMEM((2, DIM_OUT, DIM_IN), jnp.float32),
        pltpu.VMEM((DIM_OUT, DIM_IN), jnp.bfloat16),
        pltpu.SemaphoreType.DMA((2,)),
    ]


def kernel(x, coords, weights, bias):
    xf = x.reshape(-1, DIM_IN)
    cf = coords.reshape(-1).astype(jnp.int32)

    perm = jnp.argsort(cf).astype(jnp.int32)
    inv_perm = jnp.zeros((N_TOK,), jnp.int32).at[perm].set(
        jnp.arange(N_TOK, dtype=jnp.int32))
    counts = jnp.zeros((NUM_SEGMENTS,), jnp.int32).at[cf].add(1)
    ends = jnp.cumsum(counts)
    starts = ends - counts

    m0 = _group_meta(starts, ends, counts, 0)
    m1 = _group_meta(starts, ends, counts, SEG_PER_GROUP)

    x_sorted = _row_gather(xf, perm)
    wg0 = weights[:SEG_PER_GROUP].reshape(SEG_PER_GROUP, DIM_OUT, DIM_IN)
    wg1 = weights[SEG_PER_GROUP:].reshape(SEG_PER_GROUP, DIM_OUT, DIM_IN)
    b3 = bias.reshape(NUM_SEGMENTS, 1, DIM_OUT)
    bg0 = b3[:SEG_PER_GROUP]
    bg1 = b3[SEG_PER_GROUP:]

    in0, out0spec = _specs(3)
    out0 = pl.pallas_call(
        _gmm0_body,
        grid_spec=pltpu.PrefetchScalarGridSpec(
            num_scalar_prefetch=9,
            grid=(GRID_G,),
            in_specs=in0,
            out_specs=out0spec,
            scratch_shapes=_scratch(),
        ),
        out_shape=jax.ShapeDtypeStruct((N_TOK, DIM_OUT), jnp.float32),
    )(*m0[:9], x_sorted, wg0, bg0)

    in1, out1spec = _specs(4)
    out_sorted = pl.pallas_call(
        _gmm1_body,
        grid_spec=pltpu.PrefetchScalarGridSpec(
            num_scalar_prefetch=10,
            grid=(GRID_G,),
            in_specs=in1,
            out_specs=out1spec,
            scratch_shapes=_scratch(),
        ),
        out_shape=jax.ShapeDtypeStruct((N_TOK, DIM_OUT), jnp.float32),
    )(*m1, x_sorted, wg1, bg1, out0)

    out = _row_gather(out_sorted, inv_perm)
    return out.reshape(*x.shape[:-1], DIM_OUT)


# speedup vs baseline: 1.1009x; 1.1009x over previous
import functools

import jax
import jax.numpy as jnp
from jax import lax
from jax.experimental import pallas as pl
from jax.experimental.pallas import tpu as pltpu
from jax.experimental.pallas import tpu_sc as plsc

DIM_IN = 2048
DIM_OUT = 2048
NUM_SEGMENTS = 16
N_TOK = 4096

TM = 256
M_PAD = N_TOK + NUM_SEGMENTS * TM
NBP = M_PAD // TM

_NC = 2
_NS = 16
_NW = _NC * _NS
_CH = 32


def _row_gather(table, idx):
    n_out = idx.shape[0]
    ncols = table.shape[1]
    rows_per_w = n_out // _NW
    nchunk = rows_per_w // _CH
    mesh = plsc.VectorSubcoreMesh(core_axis_name="c", subcore_axis_name="s")

    @functools.partial(
        pl.kernel,
        mesh=mesh,
        out_type=jax.ShapeDtypeStruct((n_out, ncols), jnp.float32),
        scratch_types=[
            pltpu.VMEM((_CH,), jnp.int32),
            pltpu.VMEM((_CH, ncols), jnp.float32),
            pltpu.SemaphoreType.DMA,
        ],
    )
    def k(table_hbm, idx_hbm, out_hbm, idx_v, rows_v, sem):
        wid = lax.axis_index("s") * _NC + lax.axis_index("c")
        base = wid * rows_per_w
        for c in range(nchunk):
            off = base + c * _CH
            pltpu.sync_copy(idx_hbm.at[pl.ds(off, _CH)], idx_v)
            pltpu.async_copy(table_hbm.at[idx_v], rows_v, sem).wait()
            pltpu.sync_copy(rows_v, out_hbm.at[pl.ds(off, _CH)])

    return k(table, idx)


def _gmm_body(eid_ref, bid_ref, act_ref, first_ref, nxt_ref, slot_ref,
              x_ref, w_hbm, b_ref, o_ref, wbuf, wb16, sems):
    t = pl.program_id(0)

    @pl.when(t == 0)
    def _():
        pltpu.make_async_copy(
            w_hbm.at[eid_ref[0]], wbuf.at[0], sems.at[0]).start()

    @pl.when(first_ref[t] == 1)
    def _():
        slot = slot_ref[t]
        pltpu.make_async_copy(
            w_hbm.at[eid_ref[t]], wbuf.at[slot], sems.at[slot]).wait()

        @pl.when(nxt_ref[t] >= 0)
        def _():
            pltpu.make_async_copy(
                w_hbm.at[nxt_ref[t]], wbuf.at[1 - slot], sems.at[1 - slot]
            ).start()

        wb16[...] = wbuf[slot].astype(jnp.bfloat16)

    @pl.when(act_ref[t] == 1)
    def _():
        o_ref[...] = lax.dot_general(
            x_ref[...].astype(jnp.bfloat16),
            wb16[...],
            dimension_numbers=(((1,), (1,)), ((), ())),
            preferred_element_type=jnp.float32,
        ) + b_ref[0]


def _grouped_gemm(x_sorted, w3, b3, eid, bid, act, first, nxt, slot):
    return pl.pallas_call(
        _gmm_body,
        grid_spec=pltpu.PrefetchScalarGridSpec(
            num_scalar_prefetch=6,
            grid=(NBP,),
            in_specs=[
                pl.BlockSpec((TM, DIM_IN), lambda t, *s: (s[1][t], 0)),
                pl.BlockSpec(memory_space=pl.ANY),
                pl.BlockSpec((1, 1, DIM_OUT), lambda t, *s: (s[0][t], 0, 0)),
            ],
            out_specs=pl.BlockSpec((TM, DIM_OUT), lambda t, *s: (s[1][t], 0)),
            scratch_shapes=[
                pltpu.VMEM((2, DIM_OUT, DIM_IN), jnp.float32),
                pltpu.VMEM((DIM_OUT, DIM_IN), jnp.bfloat16),
                pltpu.SemaphoreType.DMA((2,)),
            ],
        ),
        out_shape=jax.ShapeDtypeStruct((M_PAD, DIM_OUT), jnp.float32),
    )(eid, bid, act, first, nxt, slot, x_sorted, w3, b3)


def kernel(x, coords, weights, bias):
    i32 = jnp.int32
    xf = x.reshape(-1, DIM_IN)
    cf = coords.reshape(-1).astype(i32)

    perm = jnp.argsort(cf).astype(i32)
    sortpos = jnp.zeros((N_TOK,), i32).at[perm].set(
        jnp.arange(N_TOK, dtype=i32))
    counts = jnp.zeros((NUM_SEGMENTS,), i32).at[cf].add(1)
    ends = jnp.cumsum(counts)
    starts = ends - counts

    cap = ((counts + TM - 1) // TM) * TM
    cumcap = jnp.cumsum(cap)
    pstart = cumcap - cap
    total_pad = cumcap[-1]

    p = jnp.arange(M_PAD, dtype=i32)
    pe = jnp.minimum(jnp.searchsorted(cumcap, p, side="right"),
                     NUM_SEGMENTS - 1).astype(i32)
    r = jnp.minimum(p - pstart[pe], jnp.maximum(counts[pe] - 1, 0))
    srow = jnp.clip(starts[pe] + r, 0, N_TOK - 1)
    gidx = jnp.where(p < total_pad, perm[srow], 0).astype(i32)

    t_idx = jnp.arange(NBP, dtype=i32)
    eid = jnp.minimum(jnp.searchsorted(cumcap, t_idx * TM, side="right"),
                      NUM_SEGMENTS - 1).astype(i32)
    act = (t_idx * TM < total_pad).astype(i32)
    used = jnp.maximum(total_pad // TM, 1)
    bid = jnp.where(act == 1, t_idx, used - 1).astype(i32)

    e_prev = jnp.concatenate([jnp.full((1,), -1, i32), eid[:-1]])
    first = ((eid != e_prev) & (act == 1)).astype(i32)
    ordinal = jnp.cumsum(first) - 1
    slot = (ordinal % 2).astype(i32)
    n_distinct = jnp.sum(first)
    opos = jnp.where(first == 1, ordinal, NBP)
    order_e = jnp.zeros((NBP,), i32).at[opos].set(eid, mode="drop")
    nxt = jnp.where(ordinal + 1 < n_distinct,
                    order_e[jnp.clip(ordinal + 1, 0, NBP - 1)],
                    -1).astype(i32)

    gpos = (pstart[cf] + (sortpos - starts[cf])).astype(i32)

    x_sorted = _row_gather(xf, gidx)
    w3 = weights.reshape(NUM_SEGMENTS, DIM_OUT, DIM_IN)
    b3 = bias.reshape(NUM_SEGMENTS, 1, DIM_OUT)
    out_sorted = _grouped_gemm(x_sorted, w3, b3, eid, bid, act, first, nxt,
                               slot)
    out = _row_gather(out_sorted, gpos)
    return out.reshape(*x.shape[:-1], DIM_OUT)


# speedup vs baseline: 1.4554x; 1.3220x over previous
import functools

import jax
import jax.numpy as jnp
from jax import lax
from jax.experimental import pallas as pl
from jax.experimental.pallas import tpu as pltpu
from jax.experimental.pallas import tpu_sc as plsc

DIM_IN = 2048
DIM_OUT = 2048
NUM_SEGMENTS = 16
N_TOK = 4096

TM = 512
NB = N_TOK // TM
GRID = NB + NUM_SEGMENTS - 1

_NC = 2
_NS = 16
_NW = _NC * _NS
_ROWS_PER_W = N_TOK // _NW
_CH = 32
_NCHUNK = _ROWS_PER_W // _CH


def _row_gather(table, idx):
    ncols = table.shape[1]
    dty = table.dtype
    mesh = plsc.VectorSubcoreMesh(core_axis_name="c", subcore_axis_name="s")

    @functools.partial(
        pl.kernel,
        mesh=mesh,
        out_type=jax.ShapeDtypeStruct((N_TOK, ncols), dty),
        scratch_types=[
            pltpu.VMEM((_CH,), jnp.int32),
            pltpu.VMEM((_CH, ncols), dty),
            pltpu.SemaphoreType.DMA,
        ],
    )
    def k(table_hbm, idx_hbm, out_hbm, idx_v, rows_v, sem):
        wid = lax.axis_index("s") * _NC + lax.axis_index("c")
        base = wid * _ROWS_PER_W
        for c in range(_NCHUNK):
            off = base + c * _CH
            pltpu.sync_copy(idx_hbm.at[pl.ds(off, _CH)], idx_v)
            pltpu.async_copy(table_hbm.at[idx_v], rows_v, sem).wait()
            pltpu.sync_copy(rows_v, out_hbm.at[pl.ds(off, _CH)])

    return k(table, idx)


def _gmm_body(eid_ref, bid_ref, rs_ref, re_ref, first_ref, nxt_ref, slot_ref,
              x_ref, w_hbm, b_ref, o_ref, wbuf, wb16, sems):
    t = pl.program_id(0)
    rs = rs_ref[t]
    re = re_ref[t]
    slot = slot_ref[t]
    w3 = w_hbm

    @pl.when(t == 0)
    def _():
        pltpu.make_async_copy(w3.at[eid_ref[0]], wbuf.at[0], sems.at[0]).start()

    @pl.when(first_ref[t] == 1)
    def _():
        pltpu.make_async_copy(
            w3.at[eid_ref[t]], wbuf.at[slot], sems.at[slot]).wait()

        @pl.when(nxt_ref[t] >= 0)
        def _():
            pltpu.make_async_copy(
                w3.at[nxt_ref[t]], wbuf.at[1 - slot], sems.at[1 - slot]
            ).start()

        wb16[...] = wbuf[slot].astype(jnp.bfloat16)

    @pl.when(rs < re)
    def _():
        base = bid_ref[t] * TM
        rows = base + lax.broadcasted_iota(jnp.int32, (TM, 1), 0)
        mask = (rows >= rs) & (rows < re)
        acc = lax.dot_general(
            x_ref[...].astype(jnp.bfloat16),
            wb16[...],
            dimension_numbers=(((1,), (1,)), ((), ())),
            preferred_element_type=jnp.float32,
        )
        o_ref[...] = jnp.where(mask, acc + b_ref[0], o_ref[...])


def _grouped_gemm(x_sorted, weights, b3, eid, bid, rs, re, first, nxt, slot):
    return pl.pallas_call(
        _gmm_body,
        grid_spec=pltpu.PrefetchScalarGridSpec(
            num_scalar_prefetch=7,
            grid=(GRID,),
            in_specs=[
                pl.BlockSpec((TM, DIM_IN), lambda t, *s: (s[1][t], 0)),
                pl.BlockSpec(memory_space=pl.ANY),
                pl.BlockSpec((1, 1, DIM_OUT), lambda t, *s: (s[0][t], 0, 0)),
            ],
            out_specs=pl.BlockSpec((TM, DIM_OUT), lambda t, *s: (s[1][t], 0)),
            scratch_shapes=[
                pltpu.VMEM((2, DIM_OUT, DIM_IN), jnp.float32),
                pltpu.VMEM((DIM_OUT, DIM_IN), jnp.bfloat16),
                pltpu.SemaphoreType.DMA((2,)),
            ],
        ),
        out_shape=jax.ShapeDtypeStruct((N_TOK, DIM_OUT), jnp.float32),
    )(eid, bid, rs, re, first, nxt, slot, x_sorted, weights, b3)


def kernel(x, coords, weights, bias):
    xf = x.reshape(-1, DIM_IN)
    cf = coords.reshape(-1).astype(jnp.int32)

    perm = jnp.argsort(cf).astype(jnp.int32)
    inv_perm = jnp.zeros((N_TOK,), jnp.int32).at[perm].set(
        jnp.arange(N_TOK, dtype=jnp.int32))
    counts = jnp.zeros((NUM_SEGMENTS,), jnp.int32).at[cf].add(1)
    ends = jnp.cumsum(counts)
    starts = ends - counts
    first_blk = starts // TM
    tiles = jnp.where(counts > 0, (ends + TM - 1) // TM - first_blk, 0)
    inc = jnp.cumsum(tiles)
    t_idx = jnp.arange(GRID, dtype=jnp.int32)
    eid = jnp.minimum(
        jnp.searchsorted(inc, t_idx, side="right"), NUM_SEGMENTS - 1
    ).astype(jnp.int32)
    tile_off = inc - tiles
    valid = t_idx < inc[-1]
    bid = jnp.where(valid, first_blk[eid] + (t_idx - tile_off[eid]),
                    NB - 1).astype(jnp.int32)
    rs = jnp.where(valid, starts[eid], 0).astype(jnp.int32)
    re = jnp.where(valid, ends[eid], 0).astype(jnp.int32)

    e_prev = jnp.concatenate([jnp.full((1,), -1, jnp.int32), eid[:-1]])
    first = ((eid != e_prev) & valid).astype(jnp.int32)
    ordinal = jnp.cumsum(first) - 1
    slot = (ordinal % 2).astype(jnp.int32)
    n_distinct = jnp.sum(first)
    pos = jnp.where(first == 1, ordinal, GRID)
    order_e = jnp.full((GRID,), -1, jnp.int32).at[pos].set(eid, mode="drop")
    nxt = jnp.where(ordinal + 1 < n_distinct,
                    order_e[jnp.clip(ordinal + 1, 0, GRID - 1)],
                    -1).astype(jnp.int32)

    x_sorted = _row_gather(xf, perm)
    w3 = weights.reshape(NUM_SEGMENTS, DIM_OUT, DIM_IN)
    b3 = bias.reshape(NUM_SEGMENTS, 1, DIM_OUT)
    out_sorted = _grouped_gemm(x_sorted, w3, b3, eid, bid, rs, re,
                               first, nxt, slot)
    out = _row_gather(out_sorted, inv_perm)
    return out.reshape(*x.shape[:-1], DIM_OUT)


# speedup vs baseline: 1.5362x; 1.0555x over previous
import functools

import jax
import jax.numpy as jnp
from jax import lax
from jax.experimental import pallas as pl
from jax.experimental.pallas import tpu as pltpu
from jax.experimental.pallas import tpu_sc as plsc

DIM_IN = 2048
DIM_OUT = 2048
NUM_SEGMENTS = 16
N_TOK = 4096

TM = 256
NB = N_TOK // TM
GRID = NB + NUM_SEGMENTS - 1

_NC = 2
_NS = 16
_NW = _NC * _NS
_ROWS_PER_W = N_TOK // _NW
_CH = 32
_NCHUNK = _ROWS_PER_W // _CH


def _row_gather(table, idx):
    ncols = table.shape[1]
    dty = table.dtype
    mesh = plsc.VectorSubcoreMesh(core_axis_name="c", subcore_axis_name="s")

    @functools.partial(
        pl.kernel,
        mesh=mesh,
        out_type=jax.ShapeDtypeStruct((N_TOK, ncols), dty),
        scratch_types=[
            pltpu.VMEM((_CH,), jnp.int32),
            pltpu.VMEM((_CH, ncols), dty),
            pltpu.SemaphoreType.DMA,
        ],
    )
    def k(table_hbm, idx_hbm, out_hbm, idx_v, rows_v, sem):
        wid = lax.axis_index("s") * _NC + lax.axis_index("c")
        base = wid * _ROWS_PER_W
        for c in range(_NCHUNK):
            off = base + c * _CH
            pltpu.sync_copy(idx_hbm.at[pl.ds(off, _CH)], idx_v)
            pltpu.async_copy(table_hbm.at[idx_v], rows_v, sem).wait()
            pltpu.sync_copy(rows_v, out_hbm.at[pl.ds(off, _CH)])

    return k(table, idx)


def _gmm_body(eid_ref, bid_ref, rs_ref, re_ref, first_ref, nxt_ref, slot_ref,
              x_ref, w_hbm, b_ref, o_ref, wbuf, wb16, sems):
    t = pl.program_id(0)
    rs = rs_ref[t]
    re = re_ref[t]
    slot = slot_ref[t]
    w3 = w_hbm

    @pl.when(t == 0)
    def _():
        pltpu.make_async_copy(w3.at[eid_ref[0]], wbuf.at[0], sems.at[0]).start()

    @pl.when(first_ref[t] == 1)
    def _():
        pltpu.make_async_copy(
            w3.at[eid_ref[t]], wbuf.at[slot], sems.at[slot]).wait()

        @pl.when(nxt_ref[t] >= 0)
        def _():
            pltpu.make_async_copy(
                w3.at[nxt_ref[t]], wbuf.at[1 - slot], sems.at[1 - slot]
            ).start()

        wb16[...] = wbuf[slot].astype(jnp.bfloat16)

    @pl.when(rs < re)
    def _():
        base = bid_ref[t] * TM
        rows = base + lax.broadcasted_iota(jnp.int32, (TM, 1), 0)
        mask = (rows >= rs) & (rows < re)
        acc = lax.dot_general(
            x_ref[...].astype(jnp.bfloat16),
            wb16[...],
            dimension_numbers=(((1,), (1,)), ((), ())),
            preferred_element_type=jnp.float32,
        )
        o_ref[...] = jnp.where(mask, acc + b_ref[0], o_ref[...])


def _grouped_gemm(x_sorted, weights, b3, eid, bid, rs, re, first, nxt, slot):
    return pl.pallas_call(
        _gmm_body,
        grid_spec=pltpu.PrefetchScalarGridSpec(
            num_scalar_prefetch=7,
            grid=(GRID,),
            in_specs=[
                pl.BlockSpec((TM, DIM_IN), lambda t, *s: (s[1][t], 0)),
                pl.BlockSpec(memory_space=pl.ANY),
                pl.BlockSpec((1, 1, DIM_OUT), lambda t, *s: (s[0][t], 0, 0)),
            ],
            out_specs=pl.BlockSpec((TM, DIM_OUT), lambda t, *s: (s[1][t], 0)),
            scratch_shapes=[
                pltpu.VMEM((2, DIM_OUT, DIM_IN), jnp.float32),
                pltpu.VMEM((DIM_OUT, DIM_IN), jnp.bfloat16),
                pltpu.SemaphoreType.DMA((2,)),
            ],
        ),
        out_shape=jax.ShapeDtypeStruct((N_TOK, DIM_OUT), jnp.float32),
    )(eid, bid, rs, re, first, nxt, slot, x_sorted, weights, b3)


def kernel(x, coords, weights, bias):
    xf = x.reshape(-1, DIM_IN)
    cf = coords.reshape(-1).astype(jnp.int32)

    perm = jnp.argsort(cf).astype(jnp.int32)
    inv_perm = jnp.zeros((N_TOK,), jnp.int32).at[perm].set(
        jnp.arange(N_TOK, dtype=jnp.int32))
    counts = jnp.zeros((NUM_SEGMENTS,), jnp.int32).at[cf].add(1)
    ends = jnp.cumsum(counts)
    starts = ends - counts
    first_blk = starts // TM
    tiles = jnp.where(counts > 0, (ends + TM - 1) // TM - first_blk, 0)
    inc = jnp.cumsum(tiles)
    t_idx = jnp.arange(GRID, dtype=jnp.int32)
    eid = jnp.minimum(
        jnp.searchsorted(inc, t_idx, side="right"), NUM_SEGMENTS - 1
    ).astype(jnp.int32)
    tile_off = inc - tiles
    valid = t_idx < inc[-1]
    bid = jnp.where(valid, first_blk[eid] + (t_idx - tile_off[eid]),
                    NB - 1).astype(jnp.int32)
    rs = jnp.where(valid, starts[eid], 0).astype(jnp.int32)
    re = jnp.where(valid, ends[eid], 0).astype(jnp.int32)

    e_prev = jnp.concatenate([jnp.full((1,), -1, jnp.int32), eid[:-1]])
    first = ((eid != e_prev) & valid).astype(jnp.int32)
    ordinal = jnp.cumsum(first) - 1
    slot = (ordinal % 2).astype(jnp.int32)
    n_distinct = jnp.sum(first)
    pos = jnp.where(first == 1, ordinal, GRID)
    order_e = jnp.full((GRID,), -1, jnp.int32).at[pos].set(eid, mode="drop")
    nxt = jnp.where(ordinal + 1 < n_distinct,
                    order_e[jnp.clip(ordinal + 1, 0, GRID - 1)],
                    -1).astype(jnp.int32)

    x_sorted = _row_gather(xf, perm)
    w3 = weights.reshape(NUM_SEGMENTS, DIM_OUT, DIM_IN)
    b3 = bias.reshape(NUM_SEGMENTS, 1, DIM_OUT)
    out_sorted = _grouped_gemm(x_sorted, w3, b3, eid, bid, rs, re,
                               first, nxt, slot)
    out = _row_gather(out_sorted, inv_perm)
    return out.reshape(*x.shape[:-1], DIM_OUT)
